# baseline (speedup 1.0000x reference)
import jax
import jax.numpy as jnp
from jax.experimental import pallas as pl
from jax.experimental.pallas import tpu as pltpu

_NBUF = 3


def _fused_kernel(x_ref, w_ref, adj_hbm, out_ref, s_ref, buf, sems):
    i = pl.program_id(0)
    nsteps = pl.num_programs(0)
    bm = buf.shape[1]

    def issue(block, slot):
        pltpu.make_async_copy(
            adj_hbm.at[pl.ds(block * bm, bm), :],
            buf.at[slot],
            sems.at[slot],
        ).start()

    @pl.when(i == 0)
    def _():
        for b in range(_NBUF):
            issue(b, b)
        s_ref[...] = jnp.dot(
            x_ref[...].astype(jnp.bfloat16),
            w_ref[...].astype(jnp.bfloat16),
            preferred_element_type=jnp.float32,
        ).astype(jnp.bfloat16)

    slot = jax.lax.rem(i, _NBUF)
    pltpu.make_async_copy(
        adj_hbm.at[pl.ds(i * bm, bm), :],
        buf.at[slot],
        sems.at[slot],
    ).wait()
    acc = jnp.dot(
        buf[slot].astype(jnp.bfloat16),
        s_ref[...],
        preferred_element_type=jnp.float32,
    )
    out_ref[...] = jnp.maximum(acc, 0.0)

    @pl.when(i + _NBUF < nsteps)
    def _():
        issue(i + _NBUF, slot)


def kernel(input, adj, W):
    n, d_in = input.shape
    d_out = W.shape[1]

    bm = 400
    out = pl.pallas_call(
        _fused_kernel,
        grid=(n // bm,),
        in_specs=[
            pl.BlockSpec((n, d_in), lambda i: (0, 0)),
            pl.BlockSpec((d_in, d_out), lambda i: (0, 0)),
            pl.BlockSpec(memory_space=pltpu.MemorySpace.HBM),
        ],
        out_specs=pl.BlockSpec((bm, d_out), lambda i: (i, 0)),
        out_shape=jax.ShapeDtypeStruct((n, d_out), jnp.float32),
        scratch_shapes=[
            pltpu.VMEM((n, d_out), jnp.bfloat16),
            pltpu.VMEM((_NBUF, bm, n), jnp.float32),
            pltpu.SemaphoreType.DMA((_NBUF,)),
        ],
    )(input, W, adj)
    return out


# speedup vs baseline: 1.0436x; 1.0436x over previous
import jax
import jax.numpy as jnp
from jax.experimental import pallas as pl
from jax.experimental.pallas import tpu as pltpu


def _fused_kernel(x_ref, w_ref, adj_ref, out_ref, s_ref):
    @pl.when(pl.program_id(0) == 0)
    def _():
        s_ref[...] = jnp.dot(
            x_ref[...].astype(jnp.bfloat16),
            w_ref[...].astype(jnp.bfloat16),
            preferred_element_type=jnp.float32,
        ).astype(jnp.bfloat16)

    acc = jnp.dot(
        adj_ref[...].astype(jnp.bfloat16),
        s_ref[...],
        preferred_element_type=jnp.float32,
    )
    out_ref[...] = jnp.maximum(acc, 0.0)


def kernel(input, adj, W):
    n, d_in = input.shape
    d_out = W.shape[1]

    bm = 400
    out = pl.pallas_call(
        _fused_kernel,
        grid=(n // bm,),
        in_specs=[
            pl.BlockSpec((n, d_in), lambda i: (0, 0)),
            pl.BlockSpec((d_in, d_out), lambda i: (0, 0)),
            pl.BlockSpec((bm, n), lambda i: (i, 0)),
        ],
        out_specs=pl.BlockSpec((bm, d_out), lambda i: (i, 0)),
        out_shape=jax.ShapeDtypeStruct((n, d_out), jnp.float32),
        scratch_shapes=[
            pltpu.VMEM((n, d_out), jnp.bfloat16),
        ],
    )(input, W, adj)
    return out
